# baseline (speedup 1.0000x reference)
import jax
import jax.numpy as jnp
from jax import lax
from jax.experimental import pallas as pl
from jax.experimental.pallas import tpu as pltpu
from jax.experimental.pallas import tpu_sc as plsc

QK_D = 32
P2 = 256
TK = 16
MB = 16
R = 64
NW = 32


def _mm_body(q_ref, k_ref, o_ref):
    scale = QK_D ** -0.5
    for b in range(MB):
        q = q_ref[b] * scale
        o_ref[b] = lax.dot_general(q, k_ref[b], (((1,), (1,)), ((), ())),
                                   preferred_element_type=jnp.float32)


def _logits(query, key, gs):
    return pl.pallas_call(
        _mm_body,
        grid=(gs // MB,),
        in_specs=[
            pl.BlockSpec((MB, P2, QK_D), lambda t: (t, 0, 0)),
            pl.BlockSpec((MB, P2, QK_D), lambda t: (t, 0, 0)),
        ],
        out_specs=pl.BlockSpec((MB, P2, P2), lambda t: (t, 0, 0)),
        out_shape=jax.ShapeDtypeStruct((gs, P2, P2), jnp.float32),
    )(query, key)


def _merge(av, ai, bv, bi, descending):
    take = av >= bv
    mv = jnp.where(take, av, bv)
    mi = jnp.where(take, ai, bi)
    return plsc.sort_key_val(mv, mi, descending=descending)


def _sc_body(gs, b0, lg, ow, oi, buf0, buf1, wb0, wb1, ib0, ib1,
             isem0, isem1, osem0, osem1):
    c = lax.axis_index("c")
    s = lax.axis_index("s")
    wid = s * 2 + c
    bpw = gs // NW
    cpb = P2 // R
    nch = bpw * cpb
    idx_consts = [lax.iota(jnp.int32, 16) + 16 * j for j in range(16)]
    bufs = ((buf0, wb0, ib0, isem0, osem0), (buf1, wb1, ib1, isem1, osem1))

    def chunk_slices(ci):
        b = wid * bpw + ci // cpb
        r0 = (ci % cpb) * R
        return (lg.at[b, pl.ds(r0, R)],
                ow.at[b0 + b, pl.ds(r0, R)],
                oi.at[b0 + b, pl.ds(r0, R)])

    def make_row_body(buf, wbuf, ibuf):
        def row_body(r):
            pairs = []
            for j in range(16):
                v = buf[r, pl.ds(16 * j, 16)]
                pairs.append(plsc.sort_key_val(v, idx_consts[j],
                                               descending=(j % 2 == 0)))
            while len(pairs) > 1:
                pairs = [_merge(*pairs[t], *pairs[t + 1],
                                descending=((t // 2) % 2 == 0
                                            or len(pairs) == 2))
                         for t in range(0, len(pairs), 2)]
            tv, ti = pairs[0]
            e = jnp.exp(tv)
            wbuf[r] = e / jnp.sum(e)
            ibuf[r] = ti
        return row_body

    for par in (0, 1):
        buf, _, _, isem, _ = bufs[par]
        src, _, _ = chunk_slices(par)
        pltpu.async_copy(src, buf, isem)

    def pair_body(i, carry):
        for par in (0, 1):
            buf, wbuf, ibuf, isem, osem = bufs[par]
            ci = 2 * i + par
            src, wdst, idst = chunk_slices(ci)
            pltpu.make_async_copy(src, buf, isem).wait()

            @pl.when(i > 0)
            def _():
                pltpu.make_async_copy(wbuf, wdst, osem).wait()
                pltpu.make_async_copy(ibuf, idst, osem).wait()

            plsc.parallel_loop(0, R, unroll=8)(make_row_body(buf, wbuf, ibuf))
            pltpu.async_copy(wbuf, wdst, osem)
            pltpu.async_copy(ibuf, idst, osem)
            nsrc, _, _ = chunk_slices((ci + 2) % nch)
            pltpu.async_copy(nsrc, buf, isem)
        return carry

    lax.fori_loop(0, nch // 2, pair_body, 0)

    for par in (0, 1):
        buf, wbuf, ibuf, isem, osem = bufs[par]
        src, wdst, idst = chunk_slices(par)
        pltpu.make_async_copy(src, buf, isem).wait()
        pltpu.make_async_copy(wbuf, wdst, osem).wait()
        pltpu.make_async_copy(ibuf, idst, osem).wait()


def _sc_topk_into(logits, w_ref, i_ref, b0):
    gs = logits.shape[0]
    mesh = plsc.VectorSubcoreMesh(core_axis_name="c", subcore_axis_name="s")
    f = pl.kernel(
        lambda *refs: _sc_body(gs, b0, *refs),
        out_type=(),
        mesh=mesh,
        compiler_params=pltpu.CompilerParams(needs_layout_passes=False),
        scratch_types=[
            pltpu.VMEM((R, P2), jnp.float32),
            pltpu.VMEM((R, P2), jnp.float32),
            pltpu.VMEM((R, TK), jnp.float32),
            pltpu.VMEM((R, TK), jnp.float32),
            pltpu.VMEM((R, TK), jnp.int32),
            pltpu.VMEM((R, TK), jnp.int32),
            pltpu.SemaphoreType.DMA,
            pltpu.SemaphoreType.DMA,
            pltpu.SemaphoreType.DMA,
            pltpu.SemaphoreType.DMA,
        ],
    )
    f(logits, w_ref, i_ref)


GROUP_SIZES = (128, 256, 320, 320)


def kernel(query, key):
    n = query.shape[0]
    w_ref = jax.empty_ref(jax.ShapeDtypeStruct((n, P2, TK), jnp.float32))
    i_ref = jax.empty_ref(jax.ShapeDtypeStruct((n, P2, TK), jnp.int32))
    b0 = 0
    for gs in GROUP_SIZES:
        q_g = lax.slice_in_dim(query, b0, b0 + gs, axis=0)
        k_g = lax.slice_in_dim(key, b0, b0 + gs, axis=0)
        lg = _logits(q_g, k_g, gs)
        _sc_topk_into(lg, w_ref, i_ref, b0)
        b0 += gs
    return (jax.freeze(w_ref), jax.freeze(i_ref))


# speedup vs baseline: 1.0081x; 1.0081x over previous
import jax
import jax.numpy as jnp
from jax import lax
from jax.experimental import pallas as pl
from jax.experimental.pallas import tpu as pltpu
from jax.experimental.pallas import tpu_sc as plsc

QK_D = 32
P2 = 256
TK = 16
MB = 32
R = 64
NW = 32


def _mm_body(q_ref, k_ref, o_ref):
    scale = QK_D ** -0.5
    for b in range(MB):
        q = q_ref[b] * scale
        o_ref[b] = lax.dot_general(q, k_ref[b], (((1,), (1,)), ((), ())),
                                   preferred_element_type=jnp.float32)


def _logits(query, key, gs):
    return pl.pallas_call(
        _mm_body,
        grid=(gs // MB,),
        in_specs=[
            pl.BlockSpec((MB, P2, QK_D), lambda t: (t, 0, 0)),
            pl.BlockSpec((MB, P2, QK_D), lambda t: (t, 0, 0)),
        ],
        out_specs=pl.BlockSpec((MB, P2, P2), lambda t: (t, 0, 0)),
        out_shape=jax.ShapeDtypeStruct((gs, P2, P2), jnp.float32),
    )(query, key)


def _merge(av, ai, bv, bi, descending):
    take = av >= bv
    mv = jnp.where(take, av, bv)
    mi = jnp.where(take, ai, bi)
    return plsc.sort_key_val(mv, mi, descending=descending)


def _sc_body(gs, b0, lg, ow, oi, buf0, buf1, wb0, wb1, ib0, ib1,
             isem0, isem1, osem0, osem1):
    c = lax.axis_index("c")
    s = lax.axis_index("s")
    wid = s * 2 + c
    bpw = gs // NW
    cpb = P2 // R
    nch = bpw * cpb
    idx_consts = [lax.iota(jnp.int32, 16) + 16 * j for j in range(16)]
    bufs = ((buf0, wb0, ib0, isem0, osem0), (buf1, wb1, ib1, isem1, osem1))

    def chunk_slices(ci):
        b = wid * bpw + ci // cpb
        r0 = (ci % cpb) * R
        return (lg.at[b, pl.ds(r0, R)],
                ow.at[b0 + b, pl.ds(r0, R)],
                oi.at[b0 + b, pl.ds(r0, R)])

    def make_row_body(buf, wbuf, ibuf):
        def row_body(r):
            pairs = []
            for j in range(16):
                v = buf[r, pl.ds(16 * j, 16)]
                pairs.append(plsc.sort_key_val(v, idx_consts[j],
                                               descending=(j % 2 == 0)))
            while len(pairs) > 1:
                pairs = [_merge(*pairs[t], *pairs[t + 1],
                                descending=((t // 2) % 2 == 0
                                            or len(pairs) == 2))
                         for t in range(0, len(pairs), 2)]
            tv, ti = pairs[0]
            e = jnp.exp(tv)
            wbuf[r] = e / jnp.sum(e)
            ibuf[r] = ti
        return row_body

    for par in (0, 1):
        buf, _, _, isem, _ = bufs[par]
        src, _, _ = chunk_slices(par)
        pltpu.async_copy(src, buf, isem)

    def pair_body(i, carry):
        for par in (0, 1):
            buf, wbuf, ibuf, isem, osem = bufs[par]
            ci = 2 * i + par
            src, wdst, idst = chunk_slices(ci)
            pltpu.make_async_copy(src, buf, isem).wait()

            @pl.when(i > 0)
            def _():
                pltpu.make_async_copy(wbuf, wdst, osem).wait()
                pltpu.make_async_copy(ibuf, idst, osem).wait()

            plsc.parallel_loop(0, R, unroll=8)(make_row_body(buf, wbuf, ibuf))
            pltpu.async_copy(wbuf, wdst, osem)
            pltpu.async_copy(ibuf, idst, osem)
            nsrc, _, _ = chunk_slices((ci + 2) % nch)
            pltpu.async_copy(nsrc, buf, isem)
        return carry

    lax.fori_loop(0, nch // 2, pair_body, 0)

    for par in (0, 1):
        buf, wbuf, ibuf, isem, osem = bufs[par]
        src, wdst, idst = chunk_slices(par)
        pltpu.make_async_copy(src, buf, isem).wait()
        pltpu.make_async_copy(wbuf, wdst, osem).wait()
        pltpu.make_async_copy(ibuf, idst, osem).wait()


def _sc_topk_into(logits, w_ref, i_ref, b0):
    gs = logits.shape[0]
    mesh = plsc.VectorSubcoreMesh(core_axis_name="c", subcore_axis_name="s")
    f = pl.kernel(
        lambda *refs: _sc_body(gs, b0, *refs),
        out_type=(),
        mesh=mesh,
        compiler_params=pltpu.CompilerParams(needs_layout_passes=False),
        scratch_types=[
            pltpu.VMEM((R, P2), jnp.float32),
            pltpu.VMEM((R, P2), jnp.float32),
            pltpu.VMEM((R, TK), jnp.float32),
            pltpu.VMEM((R, TK), jnp.float32),
            pltpu.VMEM((R, TK), jnp.int32),
            pltpu.VMEM((R, TK), jnp.int32),
            pltpu.SemaphoreType.DMA,
            pltpu.SemaphoreType.DMA,
            pltpu.SemaphoreType.DMA,
            pltpu.SemaphoreType.DMA,
        ],
    )
    f(logits, w_ref, i_ref)


GROUP_SIZES = (256, 256, 256, 256)


def kernel(query, key):
    n = query.shape[0]
    w_ref = jax.empty_ref(jax.ShapeDtypeStruct((n, P2, TK), jnp.float32))
    i_ref = jax.empty_ref(jax.ShapeDtypeStruct((n, P2, TK), jnp.int32))
    b0 = 0
    for gs in GROUP_SIZES:
        q_g = lax.slice_in_dim(query, b0, b0 + gs, axis=0)
        k_g = lax.slice_in_dim(key, b0, b0 + gs, axis=0)
        lg = _logits(q_g, k_g, gs)
        _sc_topk_into(lg, w_ref, i_ref, b0)
        b0 += gs
    return (jax.freeze(w_ref), jax.freeze(i_ref))
